# Initial kernel scaffold; baseline (speedup 1.0000x reference)
import jax
import jax.numpy as jnp
from jax import lax
from jax.experimental import pallas as pl
from jax.experimental.pallas import tpu as pltpu
from jax.experimental.pallas import tpu_sc as plsc

N = 10000
D = 128
DP = 136
E = 320000
NSUB = 16
EPT = E // NSUB
K = 80
CHUNKS = EPT // K
RPT = N // NSUB


def _sc_body(xp, edges, zblk, out, idx_s, idx_d, rows, acc, sem):
    c = lax.axis_index("c")
    s = lax.axis_index("s")
    r0 = s * RPT
    pltpu.sync_copy(zblk, acc.at[pl.ds(r0, RPT), :])
    plsc.subcore_barrier()
    ebase = s * EPT

    def chunk(j, carry):
        base = ebase + j * K
        pltpu.sync_copy(edges.at[c, 0, pl.ds(base, K)], idx_s)
        pltpu.async_copy(xp.at[idx_s], rows, sem).wait()
        pltpu.sync_copy(edges.at[c, 1, pl.ds(base, K)], idx_d)
        pltpu.sync_copy(rows, acc.at[idx_d], add=True)
        return carry

    lax.fori_loop(0, CHUNKS, chunk, 0)
    plsc.subcore_barrier()
    pltpu.sync_copy(acc.at[pl.ds(r0, RPT), :], out.at[c, pl.ds(r0, RPT), :])


_sc_agg = pl.kernel(
    _sc_body,
    out_type=jax.ShapeDtypeStruct((2, N, DP), jnp.float32),
    mesh=plsc.VectorSubcoreMesh(core_axis_name="c", subcore_axis_name="s"),
    scratch_types=[
        pltpu.VMEM((K,), jnp.int32),
        pltpu.VMEM((K,), jnp.int32),
        pltpu.VMEM((K, DP), jnp.float32),
        pltpu.VMEM_SHARED((N, DP), jnp.float32),
        pltpu.SemaphoreType.DMA,
    ],
)


def _tc_body(a0, d0, a1, d1, w0, w1, o):
    s0 = 0.5 / jnp.maximum(d0[...], 1.0)
    s1 = 0.5 / jnp.maximum(d1[...], 1.0)
    acc = jnp.dot(a0[...] * s0, w0[...], preferred_element_type=jnp.float32,
                  precision=lax.Precision.HIGHEST)
    acc = acc + jnp.dot(a1[...] * s1, w1[...], preferred_element_type=jnp.float32,
                        precision=lax.Precision.HIGHEST)
    o[...] = acc


_BN = 400

_tc_fin = pl.pallas_call(
    _tc_body,
    grid=(N // _BN,),
    in_specs=[
        pl.BlockSpec((_BN, D), lambda i: (i, 0)),
        pl.BlockSpec((_BN, 1), lambda i: (i, 0)),
        pl.BlockSpec((_BN, D), lambda i: (i, 0)),
        pl.BlockSpec((_BN, 1), lambda i: (i, 0)),
        pl.BlockSpec((D, D), lambda i: (0, 0)),
        pl.BlockSpec((D, D), lambda i: (0, 0)),
    ],
    out_specs=pl.BlockSpec((_BN, D), lambda i: (i, 0)),
    out_shape=jax.ShapeDtypeStruct((N, D), jnp.float32),
)


def kernel(x, edge_index_mp0, edge_index_mp1, W_mp0, W_mp1):
    xp = jnp.concatenate(
        [x, jnp.ones((N, 1), x.dtype), jnp.zeros((N, DP - D - 1), x.dtype)],
        axis=1)
    edges = jnp.stack([edge_index_mp0, edge_index_mp1])
    zblk = jnp.zeros((RPT, DP), jnp.float32)
    agg = _sc_agg(xp, edges, zblk)
    a0 = agg[0, :, :D]
    d0 = agg[0, :, D:D + 1]
    a1 = agg[1, :, :D]
    d1 = agg[1, :, D:D + 1]
    return _tc_fin(a0, d0, a1, d1, W_mp0, W_mp1)


# speedup vs baseline: 4.9351x; 4.9351x over previous
import jax
import jax.numpy as jnp
from jax import lax
from jax.experimental import pallas as pl
from jax.experimental.pallas import tpu as pltpu
from jax.experimental.pallas import tpu_sc as plsc

N = 10000
D = 128
DP = 136
E = 320000
NSUB = 16
EPT = E // NSUB
K = 80
CHUNKS = EPT // K
RPT = N // NSUB


def _sc_body(xp, edges, zblk, out, idx_s, idx_d, rows, acc, sem):
    c = lax.axis_index("c")
    s = lax.axis_index("s")
    r0 = s * RPT
    pltpu.sync_copy(zblk, acc.at[pl.ds(r0, RPT), :])
    plsc.subcore_barrier()
    ebase = s * EPT

    def chunk(j, carry):
        base = ebase + j * K
        pltpu.sync_copy(edges.at[c, 0, pl.ds(base, K)], idx_s)
        pltpu.async_copy(xp.at[idx_s], rows, sem).wait()
        pltpu.sync_copy(edges.at[c, 1, pl.ds(base, K)], idx_d)
        pltpu.sync_copy(rows, acc.at[idx_d], add=True)
        return carry

    lax.fori_loop(0, CHUNKS, chunk, 0)
    plsc.subcore_barrier()
    pltpu.sync_copy(acc.at[pl.ds(r0, RPT), :], out.at[c, pl.ds(r0, RPT), :])


_sc_agg = pl.kernel(
    _sc_body,
    out_type=jax.ShapeDtypeStruct((2, N, DP), jnp.float32),
    mesh=plsc.VectorSubcoreMesh(core_axis_name="c", subcore_axis_name="s"),
    compiler_params=pltpu.CompilerParams(use_tc_tiling_on_sc=False),
    scratch_types=[
        pltpu.VMEM((K,), jnp.int32),
        pltpu.VMEM((K,), jnp.int32),
        pltpu.VMEM((K, DP), jnp.float32),
        pltpu.VMEM_SHARED((N, DP), jnp.float32),
        pltpu.SemaphoreType.DMA,
    ],
)


def _tc_body(a0, d0, a1, d1, w0, w1, o):
    s0 = 0.5 / jnp.maximum(d0[...], 1.0)
    s1 = 0.5 / jnp.maximum(d1[...], 1.0)
    acc = jnp.dot(a0[...] * s0, w0[...], preferred_element_type=jnp.float32,
                  precision=lax.Precision.HIGHEST)
    acc = acc + jnp.dot(a1[...] * s1, w1[...], preferred_element_type=jnp.float32,
                        precision=lax.Precision.HIGHEST)
    o[...] = acc


_BN = 400

_tc_fin = pl.pallas_call(
    _tc_body,
    grid=(N // _BN,),
    in_specs=[
        pl.BlockSpec((_BN, D), lambda i: (i, 0)),
        pl.BlockSpec((_BN, 1), lambda i: (i, 0)),
        pl.BlockSpec((_BN, D), lambda i: (i, 0)),
        pl.BlockSpec((_BN, 1), lambda i: (i, 0)),
        pl.BlockSpec((D, D), lambda i: (0, 0)),
        pl.BlockSpec((D, D), lambda i: (0, 0)),
    ],
    out_specs=pl.BlockSpec((_BN, D), lambda i: (i, 0)),
    out_shape=jax.ShapeDtypeStruct((N, D), jnp.float32),
)


def kernel(x, edge_index_mp0, edge_index_mp1, W_mp0, W_mp1):
    xp = jnp.concatenate(
        [x, jnp.ones((N, 1), x.dtype), jnp.zeros((N, DP - D - 1), x.dtype)],
        axis=1)
    edges = jnp.stack([edge_index_mp0, edge_index_mp1])
    zblk = jnp.zeros((RPT, DP), jnp.float32)
    agg = _sc_agg(xp, edges, zblk)
    a0 = agg[0, :, :D]
    d0 = agg[0, :, D:D + 1]
    a1 = agg[1, :, :D]
    d1 = agg[1, :, D:D + 1]
    return _tc_fin(a0, d0, a1, d1, W_mp0, W_mp1)


# speedup vs baseline: 8.0865x; 1.6386x over previous
import jax
import jax.numpy as jnp
from jax import lax
from jax.experimental import pallas as pl
from jax.experimental.pallas import tpu as pltpu
from jax.experimental.pallas import tpu_sc as plsc

N = 10000
D = 128
DP = 136
E = 320000
NSUB = 16
EPT = E // NSUB
K = 80
CHUNKS = EPT // K
NB = 2
NITER = CHUNKS // NB
RPT = N // NSUB
L = 16


def _sc_body(xp, epk, zblk, out, pk2, is0, is1, id0, id1,
             rows0, rows1, g0, g1, s0, s1, acc):
    rows = [rows0, rows1]
    idx_s = [is0, is1]
    idx_d = [id0, id1]
    gs = [g0, g1]
    ss = [s0, s1]
    c = lax.axis_index("c")
    s = lax.axis_index("s")
    r0 = s * RPT
    pltpu.sync_copy(zblk, acc.at[pl.ds(r0, RPT), :])
    pltpu.sync_copy(epk.at[c, s], pk2)
    plsc.subcore_barrier()

    def unpack(j, b):
        for v in range(K // L):
            p = pk2[j, pl.ds(v * L, L)]
            idx_s[b][pl.ds(v * L, L)] = lax.bitwise_and(p, jnp.int32(0xFFFF))
            idx_d[b][pl.ds(v * L, L)] = lax.shift_right_logical(p, jnp.int32(16))

    def body(i, carry):
        for b in range(NB):
            j = i * NB + b

            @pl.when(i > 0)
            def _():
                pltpu.make_async_copy(rows[b], acc.at[idx_d[b]], ss[b]).wait()

            unpack(j, b)
            pltpu.async_copy(xp.at[idx_s[b]], rows[b], gs[b])
        for b in range(NB):
            pltpu.make_async_copy(xp.at[idx_s[b]], rows[b], gs[b]).wait()
            pltpu.async_copy(rows[b], acc.at[idx_d[b]], ss[b], add=True)
        return carry

    lax.fori_loop(0, NITER, body, 0)
    for b in range(NB):
        pltpu.make_async_copy(rows[b], acc.at[idx_d[b]], ss[b]).wait()
    plsc.subcore_barrier()
    pltpu.sync_copy(acc.at[pl.ds(r0, RPT), :], out.at[c, pl.ds(r0, RPT), :])


_sc_agg = pl.kernel(
    _sc_body,
    out_type=jax.ShapeDtypeStruct((2, N, DP), jnp.float32),
    mesh=plsc.VectorSubcoreMesh(core_axis_name="c", subcore_axis_name="s"),
    compiler_params=pltpu.CompilerParams(use_tc_tiling_on_sc=False),
    scratch_types=[
        pltpu.VMEM((CHUNKS, K), jnp.int32),
        pltpu.VMEM((K,), jnp.int32),
        pltpu.VMEM((K,), jnp.int32),
        pltpu.VMEM((K,), jnp.int32),
        pltpu.VMEM((K,), jnp.int32),
        pltpu.VMEM((K, DP), jnp.float32),
        pltpu.VMEM((K, DP), jnp.float32),
        pltpu.SemaphoreType.DMA,
        pltpu.SemaphoreType.DMA,
        pltpu.SemaphoreType.DMA,
        pltpu.SemaphoreType.DMA,
        pltpu.VMEM_SHARED((N, DP), jnp.float32),
    ],
)


def _tc_body(a0, d0, a1, d1, w0, w1, o):
    s0 = 0.5 / jnp.maximum(d0[...], 1.0)
    s1 = 0.5 / jnp.maximum(d1[...], 1.0)
    acc = jnp.dot(a0[...] * s0, w0[...], preferred_element_type=jnp.float32,
                  precision=lax.Precision.HIGHEST)
    acc = acc + jnp.dot(a1[...] * s1, w1[...], preferred_element_type=jnp.float32,
                        precision=lax.Precision.HIGHEST)
    o[...] = acc


_BN = 400

_tc_fin = pl.pallas_call(
    _tc_body,
    grid=(N // _BN,),
    in_specs=[
        pl.BlockSpec((_BN, D), lambda i: (i, 0)),
        pl.BlockSpec((_BN, 1), lambda i: (i, 0)),
        pl.BlockSpec((_BN, D), lambda i: (i, 0)),
        pl.BlockSpec((_BN, 1), lambda i: (i, 0)),
        pl.BlockSpec((D, D), lambda i: (0, 0)),
        pl.BlockSpec((D, D), lambda i: (0, 0)),
    ],
    out_specs=pl.BlockSpec((_BN, D), lambda i: (i, 0)),
    out_shape=jax.ShapeDtypeStruct((N, D), jnp.float32),
)


def kernel(x, edge_index_mp0, edge_index_mp1, W_mp0, W_mp1):
    xp = jnp.concatenate(
        [x, jnp.ones((N, 1), x.dtype), jnp.zeros((N, DP - D - 1), x.dtype)],
        axis=1)
    pk0 = edge_index_mp0[0] | (edge_index_mp0[1] << 16)
    pk1 = edge_index_mp1[0] | (edge_index_mp1[1] << 16)
    epk = jnp.stack([pk0, pk1]).reshape(2, NSUB, CHUNKS, K)
    zblk = jnp.zeros((RPT, DP), jnp.float32)
    agg = _sc_agg(xp, epk, zblk)
    a0 = agg[0, :, :D]
    d0 = agg[0, :, D:D + 1]
    a1 = agg[1, :, :D]
    d1 = agg[1, :, D:D + 1]
    return _tc_fin(a0, d0, a1, d1, W_mp0, W_mp1)


# speedup vs baseline: 8.6157x; 1.0654x over previous
import jax
import jax.numpy as jnp
from jax import lax
from jax.experimental import pallas as pl
from jax.experimental.pallas import tpu as pltpu
from jax.experimental.pallas import tpu_sc as plsc

N = 10000
D = 128
DP = 136
E = 320000
NSUB = 16
EPT = E // NSUB
K = 128
CHUNKS = EPT // K
KT = EPT - CHUNKS * K
NB = 2
NITER = CHUNKS // NB
RPT = N // NSUB
L = 16


def _sc_body(xp, epk, zblk, out, pk0b, pk1b, pkt, is0, is1, id0, id1,
             ist, idt, rows0, rows1, rowst,
             g0, g1, s0, s1, p0, p1, pt, acc):
    pk = [pk0b, pk1b]
    rows = [rows0, rows1]
    idx_s = [is0, is1]
    idx_d = [id0, id1]
    gs = [g0, g1]
    ss = [s0, s1]
    ps = [p0, p1]
    c = lax.axis_index("c")
    s = lax.axis_index("s")
    r0 = s * RPT

    def unpack(pkb, isb, idb, cnt):
        for v in range(cnt // L):
            p = pkb[pl.ds(v * L, L)]
            isb[pl.ds(v * L, L)] = lax.bitwise_and(p, jnp.int32(0xFFFF))
            idb[pl.ds(v * L, L)] = lax.shift_right_logical(p, jnp.int32(16))

    pltpu.async_copy(epk.at[c, s, pl.ds(0, K)], pk[0], ps[0])
    pltpu.async_copy(epk.at[c, s, pl.ds(K, K)], pk[1], ps[1])
    pltpu.async_copy(epk.at[c, s, pl.ds(CHUNKS * K, KT)], pkt, pt)
    pltpu.sync_copy(zblk, acc.at[pl.ds(r0, RPT), :])
    plsc.subcore_barrier()

    def body(i, carry):
        for b in range(NB):
            j = i * NB + b

            @pl.when(i > 0)
            def _():
                pltpu.make_async_copy(rows[b], acc.at[idx_d[b]], ss[b]).wait()

            pltpu.make_async_copy(epk.at[c, s, pl.ds(j * K, K)], pk[b],
                                  ps[b]).wait()
            unpack(pk[b], idx_s[b], idx_d[b], K)
            jn = jnp.minimum(j + NB, CHUNKS - 1)
            pltpu.async_copy(epk.at[c, s, pl.ds(jn * K, K)], pk[b], ps[b])
            pltpu.async_copy(xp.at[idx_s[b]], rows[b], gs[b])
        for b in range(NB):
            pltpu.make_async_copy(xp.at[idx_s[b]], rows[b], gs[b]).wait()
            pltpu.async_copy(rows[b], acc.at[idx_d[b]], ss[b], add=True)
        return carry

    lax.fori_loop(0, NITER, body, 0)
    for b in range(NB):
        pltpu.make_async_copy(rows[b], acc.at[idx_d[b]], ss[b]).wait()
        pltpu.make_async_copy(epk.at[c, s, pl.ds(0, K)], pk[b], ps[b]).wait()
    pltpu.make_async_copy(epk.at[c, s, pl.ds(CHUNKS * K, KT)], pkt, pt).wait()
    unpack(pkt, ist, idt, KT)
    pltpu.async_copy(xp.at[ist], rowst, g0).wait()
    pltpu.async_copy(rowst, acc.at[idt], s0, add=True).wait()
    plsc.subcore_barrier()
    pltpu.sync_copy(acc.at[pl.ds(r0, RPT), :], out.at[c, pl.ds(r0, RPT), :])


_sc_agg = pl.kernel(
    _sc_body,
    out_type=jax.ShapeDtypeStruct((2, N, DP), jnp.float32),
    mesh=plsc.VectorSubcoreMesh(core_axis_name="c", subcore_axis_name="s"),
    compiler_params=pltpu.CompilerParams(use_tc_tiling_on_sc=False),
    scratch_types=[
        pltpu.VMEM((K,), jnp.int32),
        pltpu.VMEM((K,), jnp.int32),
        pltpu.VMEM((KT,), jnp.int32),
        pltpu.VMEM((K,), jnp.int32),
        pltpu.VMEM((K,), jnp.int32),
        pltpu.VMEM((K,), jnp.int32),
        pltpu.VMEM((K,), jnp.int32),
        pltpu.VMEM((KT,), jnp.int32),
        pltpu.VMEM((KT,), jnp.int32),
        pltpu.VMEM((K, DP), jnp.float32),
        pltpu.VMEM((K, DP), jnp.float32),
        pltpu.VMEM((KT, DP), jnp.float32),
        pltpu.SemaphoreType.DMA,
        pltpu.SemaphoreType.DMA,
        pltpu.SemaphoreType.DMA,
        pltpu.SemaphoreType.DMA,
        pltpu.SemaphoreType.DMA,
        pltpu.SemaphoreType.DMA,
        pltpu.SemaphoreType.DMA,
        pltpu.VMEM_SHARED((N, DP), jnp.float32),
    ],
)


def _tc_body(a0, d0, a1, d1, w0, w1, o):
    s0 = 0.5 / jnp.maximum(d0[...], 1.0)
    s1 = 0.5 / jnp.maximum(d1[...], 1.0)
    acc = jnp.dot(a0[...] * s0, w0[...], preferred_element_type=jnp.float32,
                  precision=lax.Precision.HIGHEST)
    acc = acc + jnp.dot(a1[...] * s1, w1[...], preferred_element_type=jnp.float32,
                        precision=lax.Precision.HIGHEST)
    o[...] = acc


_BN = 400

_tc_fin = pl.pallas_call(
    _tc_body,
    grid=(N // _BN,),
    in_specs=[
        pl.BlockSpec((_BN, D), lambda i: (i, 0)),
        pl.BlockSpec((_BN, 1), lambda i: (i, 0)),
        pl.BlockSpec((_BN, D), lambda i: (i, 0)),
        pl.BlockSpec((_BN, 1), lambda i: (i, 0)),
        pl.BlockSpec((D, D), lambda i: (0, 0)),
        pl.BlockSpec((D, D), lambda i: (0, 0)),
    ],
    out_specs=pl.BlockSpec((_BN, D), lambda i: (i, 0)),
    out_shape=jax.ShapeDtypeStruct((N, D), jnp.float32),
)


def kernel(x, edge_index_mp0, edge_index_mp1, W_mp0, W_mp1):
    xp = jnp.concatenate(
        [x, jnp.ones((N, 1), x.dtype), jnp.zeros((N, DP - D - 1), x.dtype)],
        axis=1)
    pk0 = edge_index_mp0[0] | (edge_index_mp0[1] << 16)
    pk1 = edge_index_mp1[0] | (edge_index_mp1[1] << 16)
    epk = jnp.stack([pk0, pk1]).reshape(2, NSUB, EPT)
    zblk = jnp.zeros((RPT, DP), jnp.float32)
    agg = _sc_agg(xp, epk, zblk)
    a0 = agg[0, :, :D]
    d0 = agg[0, :, D:D + 1]
    a1 = agg[1, :, :D]
    d1 = agg[1, :, D:D + 1]
    return _tc_fin(a0, d0, a1, d1, W_mp0, W_mp1)


# speedup vs baseline: 8.8964x; 1.0326x over previous
import jax
import jax.numpy as jnp
from jax import lax
from jax.experimental import pallas as pl
from jax.experimental.pallas import tpu as pltpu
from jax.experimental.pallas import tpu_sc as plsc

N = 10000
D = 128
DP = 136
E = 320000
NSUB = 16
EPT = E // NSUB
K = 128
CHUNKS = EPT // K
KT = EPT - CHUNKS * K
NB = 2
NITER = CHUNKS // NB
RPT = N // NSUB
L = 16


def _sc_body(xp, epk, zblk, out0, out1, pk0b, pk1b, pkt, is0, is1, id0, id1,
             ist, idt, rows0, rows1, rowst,
             g0, g1, s0, s1, p0, p1, pt, acc):
    pk = [pk0b, pk1b]
    rows = [rows0, rows1]
    idx_s = [is0, is1]
    idx_d = [id0, id1]
    gs = [g0, g1]
    ss = [s0, s1]
    ps = [p0, p1]
    c = lax.axis_index("c")
    s = lax.axis_index("s")
    r0 = s * RPT

    def unpack(pkb, isb, idb, cnt):
        for v in range(cnt // L):
            p = pkb[pl.ds(v * L, L)]
            isb[pl.ds(v * L, L)] = lax.bitwise_and(p, jnp.int32(0xFFFF))
            idb[pl.ds(v * L, L)] = lax.shift_right_logical(p, jnp.int32(16))

    pltpu.async_copy(epk.at[c, s, pl.ds(0, K)], pk[0], ps[0])
    pltpu.async_copy(epk.at[c, s, pl.ds(K, K)], pk[1], ps[1])
    pltpu.async_copy(epk.at[c, s, pl.ds(CHUNKS * K, KT)], pkt, pt)
    pltpu.sync_copy(zblk, acc.at[pl.ds(r0, RPT), :])
    plsc.subcore_barrier()

    def body(i, carry):
        for b in range(NB):
            j = i * NB + b

            @pl.when(i > 0)
            def _():
                pltpu.make_async_copy(rows[b], acc.at[idx_d[b]], ss[b]).wait()

            pltpu.make_async_copy(epk.at[c, s, pl.ds(j * K, K)], pk[b],
                                  ps[b]).wait()
            unpack(pk[b], idx_s[b], idx_d[b], K)
            jn = jnp.minimum(j + NB, CHUNKS - 1)
            pltpu.async_copy(epk.at[c, s, pl.ds(jn * K, K)], pk[b], ps[b])
            pltpu.async_copy(xp.at[idx_s[b]], rows[b], gs[b])
        for b in range(NB):
            pltpu.make_async_copy(xp.at[idx_s[b]], rows[b], gs[b]).wait()
            pltpu.async_copy(rows[b], acc.at[idx_d[b]], ss[b], add=True)
        return carry

    lax.fori_loop(0, NITER, body, 0)
    for b in range(NB):
        pltpu.make_async_copy(rows[b], acc.at[idx_d[b]], ss[b]).wait()
        pltpu.make_async_copy(epk.at[c, s, pl.ds(0, K)], pk[b], ps[b]).wait()
    pltpu.make_async_copy(epk.at[c, s, pl.ds(CHUNKS * K, KT)], pkt, pt).wait()
    unpack(pkt, ist, idt, KT)
    pltpu.async_copy(xp.at[ist], rowst, g0).wait()
    pltpu.async_copy(rowst, acc.at[idt], s0, add=True).wait()
    plsc.subcore_barrier()

    @pl.when(c == 0)
    def _():
        pltpu.sync_copy(acc.at[pl.ds(r0, RPT), :], out0.at[pl.ds(r0, RPT), :])

    @pl.when(c == 1)
    def _():
        pltpu.sync_copy(acc.at[pl.ds(r0, RPT), :], out1.at[pl.ds(r0, RPT), :])


_sc_agg = pl.kernel(
    _sc_body,
    out_type=[jax.ShapeDtypeStruct((N, DP), jnp.float32),
              jax.ShapeDtypeStruct((N, DP), jnp.float32)],
    mesh=plsc.VectorSubcoreMesh(core_axis_name="c", subcore_axis_name="s"),
    compiler_params=pltpu.CompilerParams(use_tc_tiling_on_sc=False),
    scratch_types=[
        pltpu.VMEM((K,), jnp.int32),
        pltpu.VMEM((K,), jnp.int32),
        pltpu.VMEM((KT,), jnp.int32),
        pltpu.VMEM((K,), jnp.int32),
        pltpu.VMEM((K,), jnp.int32),
        pltpu.VMEM((K,), jnp.int32),
        pltpu.VMEM((K,), jnp.int32),
        pltpu.VMEM((KT,), jnp.int32),
        pltpu.VMEM((KT,), jnp.int32),
        pltpu.VMEM((K, DP), jnp.float32),
        pltpu.VMEM((K, DP), jnp.float32),
        pltpu.VMEM((KT, DP), jnp.float32),
        pltpu.SemaphoreType.DMA,
        pltpu.SemaphoreType.DMA,
        pltpu.SemaphoreType.DMA,
        pltpu.SemaphoreType.DMA,
        pltpu.SemaphoreType.DMA,
        pltpu.SemaphoreType.DMA,
        pltpu.SemaphoreType.DMA,
        pltpu.VMEM_SHARED((N, DP), jnp.float32),
    ],
)


def _tc_body(a0, a1, w0, w1, o):
    v0 = a0[...]
    v1 = a1[...]
    s0 = 0.5 / jnp.maximum(v0[:, D:D + 1], 1.0)
    s1 = 0.5 / jnp.maximum(v1[:, D:D + 1], 1.0)
    acc = jnp.dot(v0[:, :D] * s0, w0[...], preferred_element_type=jnp.float32,
                  precision=lax.Precision.HIGHEST)
    acc = acc + jnp.dot(v1[:, :D] * s1, w1[...], preferred_element_type=jnp.float32,
                        precision=lax.Precision.HIGHEST)
    o[...] = acc


_BN = 400

_tc_fin = pl.pallas_call(
    _tc_body,
    grid=(N // _BN,),
    in_specs=[
        pl.BlockSpec((_BN, DP), lambda i: (i, 0)),
        pl.BlockSpec((_BN, DP), lambda i: (i, 0)),
        pl.BlockSpec((D, D), lambda i: (0, 0)),
        pl.BlockSpec((D, D), lambda i: (0, 0)),
    ],
    out_specs=pl.BlockSpec((_BN, D), lambda i: (i, 0)),
    out_shape=jax.ShapeDtypeStruct((N, D), jnp.float32),
)


def kernel(x, edge_index_mp0, edge_index_mp1, W_mp0, W_mp1):
    xp = jnp.concatenate(
        [x, jnp.ones((N, 1), x.dtype), jnp.zeros((N, DP - D - 1), x.dtype)],
        axis=1)
    pk0 = edge_index_mp0[0] | (edge_index_mp0[1] << 16)
    pk1 = edge_index_mp1[0] | (edge_index_mp1[1] << 16)
    epk = jnp.stack([pk0, pk1]).reshape(2, NSUB, EPT)
    zblk = jnp.zeros((RPT, DP), jnp.float32)
    agg0, agg1 = _sc_agg(xp, epk, zblk)
    return _tc_fin(agg0, agg1, W_mp0, W_mp1)


# speedup vs baseline: 10.6152x; 1.1932x over previous
import jax
import jax.numpy as jnp
from jax import lax
from jax.experimental import pallas as pl
from jax.experimental.pallas import tpu as pltpu
from jax.experimental.pallas import tpu_sc as plsc

N = 10000
D = 128
DP = 136
E = 320000
NSUB = 16
EPT = E // NSUB
K = 80
CHUNKS = EPT // K
NB = 4
NITER = CHUNKS // NB
NTAIL = CHUNKS - NITER * NB
RPT = N // NSUB
L = 16


def _sc_body(xp, epk, zblk, out0, out1,
             pk0b, pk1b, pk2b, pk3b, is0, is1, is2, is3,
             id0, id1, id2, id3, rows0, rows1, rows2, rows3,
             g0, g1, g2, g3, s0, s1, s2, s3, p0, p1, p2, p3, acc):
    pk = [pk0b, pk1b, pk2b, pk3b]
    rows = [rows0, rows1, rows2, rows3]
    idx_s = [is0, is1, is2, is3]
    idx_d = [id0, id1, id2, id3]
    gs = [g0, g1, g2, g3]
    ss = [s0, s1, s2, s3]
    ps = [p0, p1, p2, p3]
    c = lax.axis_index("c")
    s = lax.axis_index("s")
    r0 = s * RPT

    def unpack(pkb, isb, idb, cnt):
        for v in range(cnt // L):
            p = pkb[pl.ds(v * L, L)]
            isb[pl.ds(v * L, L)] = lax.bitwise_and(p, jnp.int32(0xFFFF))
            idb[pl.ds(v * L, L)] = lax.shift_right_logical(p, jnp.int32(16))

    for b in range(NB):
        pltpu.async_copy(epk.at[c, s, pl.ds(b * K, K)], pk[b], ps[b])
    pltpu.sync_copy(zblk, acc.at[pl.ds(r0, RPT), :])
    plsc.subcore_barrier()

    def body(i, carry):
        for b in range(NB):
            j = i * NB + b

            @pl.when(i > 0)
            def _():
                pltpu.make_async_copy(rows[b], acc.at[idx_d[b]], ss[b]).wait()

            pltpu.make_async_copy(epk.at[c, s, pl.ds(j * K, K)], pk[b],
                                  ps[b]).wait()
            unpack(pk[b], idx_s[b], idx_d[b], K)
            jn = jnp.minimum(j + NB, CHUNKS - 1)
            pltpu.async_copy(epk.at[c, s, pl.ds(jn * K, K)], pk[b], ps[b])
            pltpu.async_copy(xp.at[idx_s[b]], rows[b], gs[b])
        for b in range(NB):
            pltpu.make_async_copy(xp.at[idx_s[b]], rows[b], gs[b]).wait()
            pltpu.async_copy(rows[b], acc.at[idx_d[b]], ss[b], add=True)
        return carry

    lax.fori_loop(0, NITER, body, 0)
    for b in range(NB):
        pltpu.make_async_copy(rows[b], acc.at[idx_d[b]], ss[b]).wait()
        pltpu.make_async_copy(epk.at[c, s, pl.ds(0, K)], pk[b], ps[b]).wait()
    for t in range(NTAIL):
        j = NITER * NB + t
        pltpu.async_copy(epk.at[c, s, pl.ds(j * K, K)], pk[t], ps[t]).wait()
        unpack(pk[t], idx_s[t], idx_d[t], K)
        pltpu.async_copy(xp.at[idx_s[t]], rows[t], gs[t]).wait()
        pltpu.async_copy(rows[t], acc.at[idx_d[t]], ss[t], add=True).wait()
    plsc.subcore_barrier()

    @pl.when(c == 0)
    def _():
        pltpu.sync_copy(acc.at[pl.ds(r0, RPT), :], out0.at[pl.ds(r0, RPT), :])

    @pl.when(c == 1)
    def _():
        pltpu.sync_copy(acc.at[pl.ds(r0, RPT), :], out1.at[pl.ds(r0, RPT), :])


_sc_agg = pl.kernel(
    _sc_body,
    out_type=[jax.ShapeDtypeStruct((N, DP), jnp.float32),
              jax.ShapeDtypeStruct((N, DP), jnp.float32)],
    mesh=plsc.VectorSubcoreMesh(core_axis_name="c", subcore_axis_name="s"),
    compiler_params=pltpu.CompilerParams(use_tc_tiling_on_sc=False),
    scratch_types=[
        pltpu.VMEM((K,), jnp.int32),
        pltpu.VMEM((K,), jnp.int32),
        pltpu.VMEM((K,), jnp.int32),
        pltpu.VMEM((K,), jnp.int32),
        pltpu.VMEM((K,), jnp.int32),
        pltpu.VMEM((K,), jnp.int32),
        pltpu.VMEM((K,), jnp.int32),
        pltpu.VMEM((K,), jnp.int32),
        pltpu.VMEM((K,), jnp.int32),
        pltpu.VMEM((K,), jnp.int32),
        pltpu.VMEM((K,), jnp.int32),
        pltpu.VMEM((K,), jnp.int32),
        pltpu.VMEM((K, DP), jnp.float32),
        pltpu.VMEM((K, DP), jnp.float32),
        pltpu.VMEM((K, DP), jnp.float32),
        pltpu.VMEM((K, DP), jnp.float32),
        pltpu.SemaphoreType.DMA,
        pltpu.SemaphoreType.DMA,
        pltpu.SemaphoreType.DMA,
        pltpu.SemaphoreType.DMA,
        pltpu.SemaphoreType.DMA,
        pltpu.SemaphoreType.DMA,
        pltpu.SemaphoreType.DMA,
        pltpu.SemaphoreType.DMA,
        pltpu.SemaphoreType.DMA,
        pltpu.SemaphoreType.DMA,
        pltpu.SemaphoreType.DMA,
        pltpu.SemaphoreType.DMA,
        pltpu.VMEM_SHARED((N, DP), jnp.float32),
    ],
)


def _tc_body(a0, a1, w0, w1, o):
    v0 = a0[...]
    v1 = a1[...]
    s0 = 0.5 / jnp.maximum(v0[:, D:D + 1], 1.0)
    s1 = 0.5 / jnp.maximum(v1[:, D:D + 1], 1.0)
    acc = jnp.dot(v0[:, :D] * s0, w0[...], preferred_element_type=jnp.float32,
                  precision=lax.Precision.HIGHEST)
    acc = acc + jnp.dot(v1[:, :D] * s1, w1[...], preferred_element_type=jnp.float32,
                        precision=lax.Precision.HIGHEST)
    o[...] = acc


_BN = 400

_tc_fin = pl.pallas_call(
    _tc_body,
    grid=(N // _BN,),
    in_specs=[
        pl.BlockSpec((_BN, DP), lambda i: (i, 0)),
        pl.BlockSpec((_BN, DP), lambda i: (i, 0)),
        pl.BlockSpec((D, D), lambda i: (0, 0)),
        pl.BlockSpec((D, D), lambda i: (0, 0)),
    ],
    out_specs=pl.BlockSpec((_BN, D), lambda i: (i, 0)),
    out_shape=jax.ShapeDtypeStruct((N, D), jnp.float32),
)


def kernel(x, edge_index_mp0, edge_index_mp1, W_mp0, W_mp1):
    xp = jnp.concatenate(
        [x, jnp.ones((N, 1), x.dtype), jnp.zeros((N, DP - D - 1), x.dtype)],
        axis=1)
    pk0 = edge_index_mp0[0] | (edge_index_mp0[1] << 16)
    pk1 = edge_index_mp1[0] | (edge_index_mp1[1] << 16)
    epk = jnp.stack([pk0, pk1]).reshape(2, NSUB, EPT)
    zblk = jnp.zeros((RPT, DP), jnp.float32)
    agg0, agg1 = _sc_agg(xp, epk, zblk)
    return _tc_fin(agg0, agg1, W_mp0, W_mp1)
